# baseline (speedup 1.0000x reference)
import functools

import jax
import jax.numpy as jnp
from jax.experimental import pallas as pl
from jax.experimental.pallas import tpu as pltpu
from jax.experimental.pallas import tpu_sc as plsc


def _sc_gather(table_fused, fused_idx):
    n = fused_idx.shape[0]
    width = table_fused.shape[1]
    window = 256
    assert n % window == 0
    mesh = plsc.VectorSubcoreMesh(core_axis_name="core", subcore_axis_name="subcore")
    idx2d = fused_idx.reshape(1, n)

    @functools.partial(
        pl.kernel,
        out_type=jax.ShapeDtypeStruct((n, width), table_fused.dtype),
        mesh=mesh,
    )
    def gather_kernel(tab_hbm, i_hbm, o_hbm):
        def body(i_vmem, o_vmem):
            pltpu.sync_copy(tab_hbm.at[i_vmem.at[0]], o_vmem)

        pltpu.emit_pipeline(
            body,
            grid=(n // window,),
            in_specs=[pl.BlockSpec((1, window), lambda i: (0, i))],
            out_specs=[pl.BlockSpec((window, width), lambda i: (i, 0))],
            core_axis_name=("core", "subcore"),
            dimension_semantics=(pltpu.PARALLEL,),
        )(i_hbm, o_hbm)

    return gather_kernel(table_fused, idx2d)


def _tc_select_linear(fused_rows, idx_col, w, b):
    n, width = fused_rows.shape
    dim = w.shape[0]
    blk = 2048
    assert n % blk == 0

    def mm_kernel(f_ref, p_ref, w_ref, b_ref, o_ref):
        odd = (p_ref[...] & 1) == 1
        e = jnp.where(odd, f_ref[:, dim:], f_ref[:, :dim])
        o_ref[...] = (
            jax.lax.dot_general(
                e,
                w_ref[...],
                (((1,), (1,)), ((), ())),
                preferred_element_type=jnp.float32,
            )
            + b_ref[...]
        )

    return pl.pallas_call(
        mm_kernel,
        grid=(n // blk,),
        in_specs=[
            pl.BlockSpec((blk, width), lambda i: (i, 0)),
            pl.BlockSpec((blk, 1), lambda i: (i, 0)),
            pl.BlockSpec((dim, dim), lambda i: (0, 0)),
            pl.BlockSpec((1, dim), lambda i: (0, 0)),
        ],
        out_specs=pl.BlockSpec((blk, dim), lambda i: (i, 0)),
        out_shape=jax.ShapeDtypeStruct((n, dim), jnp.float32),
    )(fused_rows, idx_col, w, b.reshape(1, dim))


def kernel(x, tok_embs, W, b):
    batch, seqlen = x.shape
    vocab, dim = tok_embs.shape
    idx = x.reshape(-1)
    table_fused = tok_embs.reshape(vocab // 2, 2 * dim)
    fused_rows = _sc_gather(table_fused, idx >> 1)
    z = _tc_select_linear(fused_rows, idx.reshape(-1, 1), W, b)
    return z.reshape(batch, seqlen, dim)


# speedup vs baseline: 1.1338x; 1.1338x over previous
import functools

import jax
import jax.numpy as jnp
from jax.experimental import pallas as pl
from jax.experimental.pallas import tpu as pltpu
from jax.experimental.pallas import tpu_sc as plsc


def _tc_project_table(tok_embs, w, b):
    vocab, dim = tok_embs.shape
    blk = 8000
    assert vocab % blk == 0

    def proj_kernel(e_ref, w_ref, b_ref, o_ref):
        z = (
            jax.lax.dot_general(
                e_ref[...],
                w_ref[...],
                (((1,), (1,)), ((), ())),
                preferred_element_type=jnp.float32,
            )
            + b_ref[...]
        )
        o_ref[:, :dim] = z
        o_ref[:, dim:] = jnp.zeros_like(z)

    return pl.pallas_call(
        proj_kernel,
        grid=(vocab // blk,),
        in_specs=[
            pl.BlockSpec((blk, dim), lambda i: (i, 0)),
            pl.BlockSpec((dim, dim), lambda i: (0, 0)),
            pl.BlockSpec((1, dim), lambda i: (0, 0)),
        ],
        out_specs=pl.BlockSpec((blk, 2 * dim), lambda i: (i, 0)),
        out_shape=jax.ShapeDtypeStruct((vocab, 2 * dim), jnp.float32),
    )(tok_embs, w, b.reshape(1, dim))


def _sc_gather(table_wide, idx_flat):
    n = idx_flat.shape[0]
    width = table_wide.shape[1]
    window = 256
    assert n % window == 0
    mesh = plsc.VectorSubcoreMesh(core_axis_name="core", subcore_axis_name="subcore")
    idx2d = idx_flat.reshape(1, n)

    @functools.partial(
        pl.kernel,
        out_type=jax.ShapeDtypeStruct((n, width), table_wide.dtype),
        mesh=mesh,
    )
    def gather_kernel(tab_hbm, i_hbm, o_hbm):
        def body(i_vmem, o_vmem):
            pltpu.sync_copy(tab_hbm.at[i_vmem.at[0]], o_vmem)

        pltpu.emit_pipeline(
            body,
            grid=(n // window,),
            in_specs=[pl.BlockSpec((1, window), lambda i: (0, i))],
            out_specs=[pl.BlockSpec((window, width), lambda i: (i, 0))],
            core_axis_name=("core", "subcore"),
            dimension_semantics=(pltpu.PARALLEL,),
        )(i_hbm, o_hbm)

    return gather_kernel(table_wide, idx2d)


def kernel(x, tok_embs, W, b):
    batch, seqlen = x.shape
    dim = tok_embs.shape[1]
    proj = _tc_project_table(tok_embs, W, b)
    rows = _sc_gather(proj, x.reshape(-1))
    return rows[:, :dim].reshape(batch, seqlen, dim)
